# Initial kernel scaffold; baseline (speedup 1.0000x reference)
import jax
import jax.numpy as jnp
from jax.experimental import pallas as pl


def kernel(x, gate_w, w1, b1, w2, b2):
    raise NotImplementedError("write your pallas kernel here")


# speedup vs baseline: 2.4062x; 2.4062x over previous
import functools
import math

import jax
import jax.numpy as jnp
from jax.experimental import pallas as pl
from jax.experimental.pallas import tpu as pltpu

_BR = 256
_NH = 2


def _ffn_block_kernel(bmap_ref, xs_ref, w1_ref, b1_ref, w2_ref, b2_ref, out_ref):
    h_idx = pl.program_id(1)
    x = xs_ref[...]
    h = jnp.dot(x, w1_ref[0], preferred_element_type=jnp.float32)
    h = h + b1_ref[0]
    h = 0.5 * h * (1.0 + jax.lax.erf(h * (1.0 / math.sqrt(2.0))))
    part = jnp.dot(h, w2_ref[0], preferred_element_type=jnp.float32)

    @pl.when(h_idx == 0)
    def _():
        out_ref[...] = part + b2_ref[0]

    @pl.when(h_idx != 0)
    def _():
        out_ref[...] = out_ref[...] + part


def _grouped_ffn(xs, bmap, w1, b1, w2, b2, nb, hb):
    E, D, H = w1.shape
    P = xs.shape[0]
    nh = H // hb
    grid_spec = pltpu.PrefetchScalarGridSpec(
        num_scalar_prefetch=1,
        grid=(nb, nh),
        in_specs=[
            pl.BlockSpec((_BR, D), lambda b, h, bm: (b, 0)),
            pl.BlockSpec((1, D, hb), lambda b, h, bm: (bm[b], 0, h)),
            pl.BlockSpec((1, 1, hb), lambda b, h, bm: (bm[b], 0, h)),
            pl.BlockSpec((1, hb, D), lambda b, h, bm: (bm[b], h, 0)),
            pl.BlockSpec((1, 1, D), lambda b, h, bm: (bm[b], 0, 0)),
        ],
        out_specs=pl.BlockSpec((_BR, D), lambda b, h, bm: (b, 0)),
    )
    return pl.pallas_call(
        _ffn_block_kernel,
        grid_spec=grid_spec,
        out_shape=jax.ShapeDtypeStruct((P, D), jnp.float32),
        compiler_params=pltpu.CompilerParams(
            dimension_semantics=("arbitrary", "arbitrary"),
        ),
    )(bmap, xs, w1, b1.reshape(E, 1, H), w2, b2.reshape(E, 1, D))


def kernel(x, gate_w, w1, b1, w2, b2):
    B, S, D = x.shape
    T = B * S
    E, _, H = w1.shape
    x_flat = x.reshape(T, D)

    logits = x_flat @ gate_w
    i1 = jnp.argmax(logits, axis=-1)
    v1 = jnp.max(logits, axis=-1)
    masked = jnp.where(jax.nn.one_hot(i1, E, dtype=bool), -jnp.inf, logits)
    i2 = jnp.argmax(masked, axis=-1)
    v2 = jnp.max(masked, axis=-1)
    e2 = jnp.exp(v2 - v1)
    wt1 = 1.0 / (1.0 + e2)
    wt2 = e2 / (1.0 + e2)

    e_pairs = jnp.stack([i1, i2], axis=1).reshape(-1).astype(jnp.int32)
    onehot = (e_pairs[:, None] == jnp.arange(E, dtype=jnp.int32)[None, :])
    rank = jnp.take_along_axis(
        jnp.cumsum(onehot.astype(jnp.int32), axis=0) - 1,
        e_pairs[:, None], axis=1)[:, 0]
    counts = jnp.sum(onehot, axis=0, dtype=jnp.int32)
    padded = ((counts + _BR - 1) // _BR) * _BR
    pad_cum = jnp.cumsum(padded)
    start = pad_cum - padded
    slot = start[e_pairs] + rank

    nb = (2 * T) // _BR + E
    P = nb * _BR
    row_token = jnp.zeros((P,), jnp.int32).at[slot].set(
        jnp.arange(2 * T, dtype=jnp.int32) // 2)
    bstart = jnp.arange(nb, dtype=jnp.int32) * _BR
    bmap = jnp.minimum(
        jnp.searchsorted(pad_cum, bstart, side="right"), E - 1
    ).astype(jnp.int32)
    pos = slot.reshape(T, 2)

    xs = x_flat[row_token]
    contrib = _grouped_ffn(xs, bmap, w1, b1, w2, b2, nb, H // _NH)
    out = wt1[:, None] * contrib[pos[:, 0]] + wt2[:, None] * contrib[pos[:, 1]]
    return out.reshape(B, S, D)
